# baseline (speedup 1.0000x reference)
import functools

import jax
import jax.numpy as jnp
from jax.experimental import pallas as pl
from jax.experimental.pallas import tpu as pltpu

_NUM_EMBEDDINGS = 512
_DIM = 32
_COMMITMENT_COST = 0.25
_BLOCK = 4000


def _vq_block_kernel(h4_ref, cb_ref, cc_ref, q4_ref, idx_ref, loss_ref):
    h4 = h4_ref[...]
    cb = cb_ref[...]
    cc = cc_ref[0, :]
    qs, idxs, loss_parts = [], [], []
    for j in range(4):
        h = h4[:, j * _DIM:(j + 1) * _DIM]
        hh = jnp.sum(h * h, axis=1, keepdims=True)
        cross2 = jax.lax.dot_general(
            h * (-2.0), cb, (((1,), (1,)), ((), ())),
            preferred_element_type=jnp.float32)
        d = (hh + cc[None, :]) + cross2
        dmin = jnp.min(d, axis=1, keepdims=True)
        iota_f = jax.lax.broadcasted_iota(
            jnp.int32, d.shape, 1).astype(jnp.float32)
        idx_f = jnp.min(jnp.where(d <= dmin, iota_f, float(_NUM_EMBEDDINGS)),
                        axis=1, keepdims=True)
        onehot = (iota_f == idx_f).astype(jnp.float32)
        qs.append(jax.lax.dot_general(
            onehot, cb, (((1,), (0,)), ((), ())),
            preferred_element_type=jnp.float32))
        idxs.append(idx_f)
        loss_parts.append(jnp.sum(dmin, axis=0, keepdims=True))
    q4_ref[...] = jnp.concatenate(qs, axis=1)
    idxc = jnp.concatenate(idxs, axis=1)
    idx_ref[...] = idxc.astype(jnp.int32)[None]
    total = loss_parts[0] + loss_parts[1] + loss_parts[2] + loss_parts[3]
    loss_ref[...] = total[None]


@functools.partial(jax.jit, static_argnames=())
def kernel(h_v_k, codebook):
    n, d = h_v_k.shape
    e = codebook.shape[0]
    cc = jnp.sum(codebook * codebook, axis=1)[None, :]
    grid = n // _BLOCK
    q4, idx, loss = pl.pallas_call(
        _vq_block_kernel,
        grid=(grid,),
        in_specs=[
            pl.BlockSpec((_BLOCK // 4, 4 * d), lambda i: (i, 0)),
            pl.BlockSpec((e, d), lambda i: (0, 0)),
            pl.BlockSpec((1, e), lambda i: (0, 0)),
        ],
        out_specs=[
            pl.BlockSpec((_BLOCK // 4, 4 * d), lambda i: (i, 0)),
            pl.BlockSpec((1, _BLOCK // 4, 4), lambda i: (i, 0, 0)),
            pl.BlockSpec((1, 1, 1), lambda i: (i, 0, 0)),
        ],
        out_shape=[
            jax.ShapeDtypeStruct((n // 4, 4 * d), jnp.float32),
            jax.ShapeDtypeStruct((grid, _BLOCK // 4, 4), jnp.int32),
            jax.ShapeDtypeStruct((grid, 1, 1), jnp.float32),
        ],
        compiler_params=pltpu.CompilerParams(
            dimension_semantics=("parallel",)),
    )(h_v_k.reshape(n // 4, 4 * d), codebook, cc)
    total_loss = jnp.sum(loss) * ((1.0 + _COMMITMENT_COST) / (n * d))
    return (q4.reshape(n, d), idx.reshape(n), total_loss)


# speedup vs baseline: 1.3787x; 1.3787x over previous
import functools

import jax
import jax.numpy as jnp
from jax.experimental import pallas as pl
from jax.experimental.pallas import tpu as pltpu

_NUM_EMBEDDINGS = 512
_DIM = 32
_COMMITMENT_COST = 0.25
_BLOCK = 4000


def _vq_block_kernel(h4_ref, cb_ref, cc_ref, q4_ref, idx_ref, loss_ref):
    h4 = h4_ref[...]
    cb = cb_ref[...]
    cc = cc_ref[0, :]
    b4 = h4.shape[0]
    h = jnp.concatenate(
        [h4[:, j * _DIM:(j + 1) * _DIM] for j in range(4)], axis=0)
    hh = jnp.sum(h * h, axis=1, keepdims=True)
    cross2 = jax.lax.dot_general(
        h * (-2.0), cb, (((1,), (1,)), ((), ())),
        preferred_element_type=jnp.float32)
    d = (hh + cc[None, :]) + cross2
    dmin = jnp.min(d, axis=1, keepdims=True)
    iota_f = jax.lax.broadcasted_iota(jnp.int32, d.shape, 1).astype(jnp.float32)
    idx_f = jnp.min(jnp.where(d <= dmin, iota_f, float(_NUM_EMBEDDINGS)),
                    axis=1, keepdims=True)
    onehot = (iota_f == idx_f).astype(jnp.float32)
    q = jax.lax.dot_general(
        onehot, cb, (((1,), (0,)), ((), ())),
        preferred_element_type=jnp.float32)
    q4_ref[...] = jnp.concatenate(
        [q[j * b4:(j + 1) * b4, :] for j in range(4)], axis=1)
    idxc = jnp.concatenate(
        [idx_f[j * b4:(j + 1) * b4, :] for j in range(4)], axis=1)
    idx_ref[...] = idxc.astype(jnp.int32)[None]
    loss_ref[...] = jnp.sum(dmin, axis=0, keepdims=True)[None]


@functools.partial(jax.jit, static_argnames=())
def kernel(h_v_k, codebook):
    n, d = h_v_k.shape
    e = codebook.shape[0]
    cc = jnp.sum(codebook * codebook, axis=1)[None, :]
    grid = n // _BLOCK
    q4, idx, loss = pl.pallas_call(
        _vq_block_kernel,
        grid=(grid,),
        in_specs=[
            pl.BlockSpec((_BLOCK // 4, 4 * d), lambda i: (i, 0)),
            pl.BlockSpec((e, d), lambda i: (0, 0)),
            pl.BlockSpec((1, e), lambda i: (0, 0)),
        ],
        out_specs=[
            pl.BlockSpec((_BLOCK // 4, 4 * d), lambda i: (i, 0)),
            pl.BlockSpec((1, _BLOCK // 4, 4), lambda i: (i, 0, 0)),
            pl.BlockSpec((1, 1, 1), lambda i: (i, 0, 0)),
        ],
        out_shape=[
            jax.ShapeDtypeStruct((n // 4, 4 * d), jnp.float32),
            jax.ShapeDtypeStruct((grid, _BLOCK // 4, 4), jnp.int32),
            jax.ShapeDtypeStruct((grid, 1, 1), jnp.float32),
        ],
        compiler_params=pltpu.CompilerParams(
            dimension_semantics=("parallel",)),
    )(h_v_k.reshape(n // 4, 4 * d), codebook, cc)
    total_loss = jnp.sum(loss) * ((1.0 + _COMMITMENT_COST) / (n * d))
    return (q4.reshape(n, d), idx.reshape(n), total_loss)


# speedup vs baseline: 1.7967x; 1.3031x over previous
import functools

import jax
import jax.numpy as jnp
from jax.experimental import pallas as pl
from jax.experimental.pallas import tpu as pltpu
from jax.experimental.layout import Format, Layout

_NUM_EMBEDDINGS = 512
_DIM = 32
_COMMITMENT_COST = 0.25
_BLOCK = 4000


def _vq_block_kernel(h_ref, cb_ref, cc_ref, q_ref, idx_ref, loss_ref):
    h = h_ref[...]
    cb = cb_ref[...]
    hh = jnp.sum(h * h, axis=1, keepdims=True)
    cc = cc_ref[0, :]
    cross2 = jax.lax.dot_general(
        h * (-2.0), cb, (((1,), (1,)), ((), ())),
        preferred_element_type=jnp.float32)
    d = (hh + cc[None, :]) + cross2
    dmin = jnp.min(d, axis=1, keepdims=True)
    iota_f = jax.lax.broadcasted_iota(jnp.int32, d.shape, 1).astype(jnp.float32)
    idx_f = jnp.min(jnp.where(d <= dmin, iota_f, float(_NUM_EMBEDDINGS)),
                    axis=1, keepdims=True)
    onehot = (iota_f == idx_f).astype(jnp.float32)
    q = jax.lax.dot_general(
        onehot, cb, (((1,), (0,)), ((), ())),
        preferred_element_type=jnp.float32)
    q_ref[...] = q
    b = idx_f.shape[0]
    idx_ref[...] = jnp.reshape(idx_f.astype(jnp.int32), (1, 8, b // 8))
    loss_ref[...] = jnp.sum(dmin, axis=0, keepdims=True)[None]


_Q_FORMAT = Format(Layout(major_to_minor=(0, 1), tiling=((8, 128),)),
                   jax.sharding.SingleDeviceSharding(jax.devices()[0]))


@functools.partial(jax.jit, static_argnames=(),
                   out_shardings=(_Q_FORMAT, None, None))
def kernel(h_v_k, codebook):
    n, d = h_v_k.shape
    e = codebook.shape[0]
    cc = jnp.sum(codebook * codebook, axis=1)[None, :]
    grid = n // _BLOCK
    q, idx, loss = pl.pallas_call(
        _vq_block_kernel,
        grid=(grid,),
        in_specs=[
            pl.BlockSpec((_BLOCK, d), lambda i: (i, 0)),
            pl.BlockSpec((e, d), lambda i: (0, 0)),
            pl.BlockSpec((1, e), lambda i: (0, 0)),
        ],
        out_specs=[
            pl.BlockSpec((_BLOCK, d), lambda i: (i, 0)),
            pl.BlockSpec((1, 8, _BLOCK // 8), lambda i: (i, 0, 0)),
            pl.BlockSpec((1, 1, 1), lambda i: (i, 0, 0)),
        ],
        out_shape=[
            jax.ShapeDtypeStruct((n, d), jnp.float32),
            jax.ShapeDtypeStruct((grid, 8, _BLOCK // 8), jnp.int32),
            jax.ShapeDtypeStruct((grid, 1, 1), jnp.float32),
        ],
        compiler_params=pltpu.CompilerParams(
            dimension_semantics=("parallel",)),
    )(h_v_k, codebook, cc)
    total_loss = jnp.sum(loss) * ((1.0 + _COMMITMENT_COST) / (n * d))
    return (q, idx.reshape(n), total_loss)


# speedup vs baseline: 1.7976x; 1.0005x over previous
import functools

import jax
import jax.numpy as jnp
from jax.experimental import pallas as pl
from jax.experimental.pallas import tpu as pltpu

_NUM_EMBEDDINGS = 512
_DIM = 32
_COMMITMENT_COST = 0.25
_BLOCK = 4000


def _vq_block_kernel(h_ref, cb_ref, cc_ref, q_ref, idx_ref, loss_ref):
    h = h_ref[...]
    cb = cb_ref[...]
    hh = jnp.sum(h * h, axis=1, keepdims=True)
    cc = cc_ref[0, :]
    cross2 = jax.lax.dot_general(
        h * (-2.0), cb, (((1,), (1,)), ((), ())),
        preferred_element_type=jnp.float32)
    d = (hh + cc[None, :]) + cross2
    dmin = jnp.min(d, axis=1, keepdims=True)
    iota_f = jax.lax.broadcasted_iota(jnp.int32, d.shape, 1).astype(jnp.float32)
    idx_f = jnp.min(jnp.where(d <= dmin, iota_f, float(_NUM_EMBEDDINGS)),
                    axis=1, keepdims=True)
    onehot = (iota_f == idx_f).astype(jnp.float32)
    q = jax.lax.dot_general(
        onehot, cb, (((1,), (0,)), ((), ())),
        preferred_element_type=jnp.float32)
    q_ref[...] = q
    b = idx_f.shape[0]
    idx_ref[...] = jnp.reshape(idx_f.astype(jnp.int32), (1, 8, b // 8))
    loss_ref[...] = jnp.sum(dmin, axis=0, keepdims=True)[None]


@functools.partial(jax.jit, static_argnames=())
def kernel(h_v_k, codebook):
    n, d = h_v_k.shape
    e = codebook.shape[0]
    cc = jnp.sum(codebook * codebook, axis=1)[None, :]
    grid = n // _BLOCK
    q, idx, loss = pl.pallas_call(
        _vq_block_kernel,
        grid=(grid,),
        in_specs=[
            pl.BlockSpec((_BLOCK, d), lambda i: (i, 0)),
            pl.BlockSpec((e, d), lambda i: (0, 0)),
            pl.BlockSpec((1, e), lambda i: (0, 0)),
        ],
        out_specs=[
            pl.BlockSpec((_BLOCK, d), lambda i: (i, 0)),
            pl.BlockSpec((1, 8, _BLOCK // 8), lambda i: (i, 0, 0)),
            pl.BlockSpec((1, 1, 1), lambda i: (i, 0, 0)),
        ],
        out_shape=[
            jax.ShapeDtypeStruct((n, d), jnp.float32),
            jax.ShapeDtypeStruct((grid, 8, _BLOCK // 8), jnp.int32),
            jax.ShapeDtypeStruct((grid, 1, 1), jnp.float32),
        ],
        compiler_params=pltpu.CompilerParams(
            dimension_semantics=("parallel",)),
    )(h_v_k, codebook, cc)
    total_loss = jnp.sum(loss) * ((1.0 + _COMMITMENT_COST) / (n * d))
    return (q, idx.reshape(n), total_loss)
